# baseline (speedup 1.0000x reference)
import jax
import jax.numpy as jnp
from jax.experimental import pallas as pl
from jax.experimental.pallas import tpu as pltpu

M_BLK = 10000


def _matmul_body(x_ref, w_ref, o_ref):
    o_ref[...] = jnp.dot(x_ref[...], w_ref[...],
                         preferred_element_type=jnp.float32)


def _content_matmul(item_content, W):
    M, K = item_content.shape
    N = W.shape[1]
    grid = (M // M_BLK,)
    return pl.pallas_call(
        _matmul_body,
        grid=grid,
        in_specs=[
            pl.BlockSpec((M_BLK, K), lambda i: (i, 0)),
            pl.BlockSpec((K, N), lambda i: (0, 0)),
        ],
        out_specs=pl.BlockSpec((M_BLK, N), lambda i: (i, 0)),
        out_shape=jax.ShapeDtypeStruct((M, N), jnp.float32),
        compiler_params=pltpu.CompilerParams(
            dimension_semantics=("arbitrary",),
            vmem_limit_bytes=110 * 1024 * 1024,
        ),
    )(item_content, W)


from jax import lax
from jax.experimental.pallas import tpu_sc as plsc

NUM_SC_CORES = 2
NUM_SC_SUBCORES = 16
NUM_WORKERS = NUM_SC_CORES * NUM_SC_SUBCORES
CH = 488


def _staged_copy(src, dst, base, nch, bufs, sin, sout):
    def wait_in(b):
        pltpu.make_async_copy(src.at[pl.ds(0, CH)], bufs[b], sin[b]).wait()

    def wait_out(b):
        pltpu.make_async_copy(bufs[b], dst.at[pl.ds(0, CH)], sout[b]).wait()

    def step(i, b):
        @pl.when(i >= 2)
        def _():
            wait_out(b)
        pltpu.async_copy(src.at[pl.ds(base + i * CH, CH)], bufs[b], sin[b])
        wait_in(b)
        pltpu.async_copy(bufs[b], dst.at[pl.ds(base + i * CH, CH)], sout[b])

    def pair(j, carry):
        step(2 * j, 0)
        step(2 * j + 1, 1)
        return carry

    lax.fori_loop(0, nch // 2, pair, 0)
    wait_out(0)
    wait_out(1)


def _sc_copy_body(p_hbm, op_hbm, b0, b1, si0, si1, so0, so1):
    wid = lax.axis_index("s") * NUM_SC_CORES + lax.axis_index("c")
    n = p_hbm.shape[0]
    rows = (n // NUM_WORKERS) // (2 * CH) * (2 * CH)
    tail = n - rows * NUM_WORKERS
    base = wid * rows
    _staged_copy(p_hbm, op_hbm, base, rows // CH, (b0, b1),
                 (si0, si1), (so0, so1))
    if tail:
        @pl.when(wid == NUM_WORKERS - 1)
        def _copy_tail():
            t = rows * NUM_WORKERS
            pltpu.async_copy(p_hbm.at[pl.ds(t, tail)],
                             b0.at[pl.ds(0, tail)], si0).wait()
            pltpu.async_copy(b0.at[pl.ds(0, tail)],
                             op_hbm.at[pl.ds(t, tail)], so0).wait()


def _sc_copy_table(Pr):
    mesh = plsc.VectorSubcoreMesh(core_axis_name="c", subcore_axis_name="s")
    fn = pl.kernel(
        _sc_copy_body,
        out_type=jax.ShapeDtypeStruct(Pr.shape, Pr.dtype),
        mesh=mesh,
        scratch_types=[
            pltpu.VMEM((CH, 128), jnp.float32),
            pltpu.VMEM((CH, 128), jnp.float32),
            pltpu.SemaphoreType.DMA,
            pltpu.SemaphoreType.DMA,
            pltpu.SemaphoreType.DMA,
            pltpu.SemaphoreType.DMA,
        ],
    )
    return fn(Pr)


def kernel(P, Q, PQ2, item_content, W):
    oP = _sc_copy_table(P.reshape(-1, 128)).reshape(P.shape)
    item_emb2 = _content_matmul(item_content, W)
    return (oP, Q, PQ2, item_emb2, W)


# speedup vs baseline: 3.6442x; 3.6442x over previous
import jax
import jax.numpy as jnp
from jax.experimental import pallas as pl
from jax.experimental.pallas import tpu as pltpu

M_BLK = 10000


def _matmul_body(x_ref, w_ref, o_ref):
    o_ref[...] = jnp.dot(x_ref[...], w_ref[...],
                         preferred_element_type=jnp.float32)


def _content_matmul(item_content, W):
    M, K = item_content.shape
    N = W.shape[1]
    grid = (M // M_BLK,)
    return pl.pallas_call(
        _matmul_body,
        grid=grid,
        in_specs=[
            pl.BlockSpec((M_BLK, K), lambda i: (i, 0)),
            pl.BlockSpec((K, N), lambda i: (0, 0)),
        ],
        out_specs=pl.BlockSpec((M_BLK, N), lambda i: (i, 0)),
        out_shape=jax.ShapeDtypeStruct((M, N), jnp.float32),
        compiler_params=pltpu.CompilerParams(
            dimension_semantics=("arbitrary",),
        ),
    )(item_content, W)


def kernel(P, Q, PQ2, item_content, W):
    item_emb2 = _content_matmul(item_content, W)
    return (P, Q, PQ2, item_emb2, W)
